# baseline (speedup 1.0000x reference)
import functools

import jax
import jax.numpy as jnp
from jax import lax
from jax.experimental import pallas as pl
from jax.experimental.pallas import tpu as pltpu
from jax.experimental.pallas import tpu_sc as plsc

N = 10000
E = 320000
D = 128
L = 16
NV = D // L
NW = 32
SPW = 320
NPAD = NW * SPW
PTR_BUF = SPW + 16
PTR_PAD = (NW - 1) * SPW + PTR_BUF
K = 128
E_PAD = E + 4 * K
OB = 8


def _body(x_hbm, ptr_hbm, idx_hbm, out_hbm, ptr_v, cidx_v, rows_v, blk_v, ptr_s,
          x_sh, rsem, isem, osem):
    _NEG_INF = jnp.full((L,), -jnp.inf, dtype=jnp.float32)
    _ZERO = jnp.zeros((L,), dtype=jnp.float32)
    cid = lax.axis_index("c")
    sid = lax.axis_index("s")
    wid = sid * 2 + cid
    s0 = wid * SPW

    pltpu.sync_copy(ptr_hbm.at[pl.ds(s0, PTR_BUF)], ptr_v)
    for b in range(PTR_BUF // L):
        vec = ptr_v[pl.ds(b * L, L)]
        for l in range(L):
            ptr_s[b * L + l] = vec[l]

    RPS = NPAD // 16
    pltpu.sync_copy(x_hbm.at[pl.ds(sid * RPS, RPS)],
                    x_sh.at[pl.ds(sid * RPS, RPS)])
    plsc.subcore_barrier()

    e0 = ptr_s[0]
    eb = (e0 // 16) * 16

    def issue_idx(c, buf):
        pltpu.async_copy(
            idx_hbm.at[pl.ds(eb + c * K, K)], cidx_v.at[buf], isem)

    def wait_idx(buf):
        pltpu.make_async_copy(
            idx_hbm.at[pl.ds(eb, K)], cidx_v.at[buf], isem).wait()

    def issue_rows(buf, src):
        pltpu.async_copy(src.at[cidx_v.at[buf]], rows_v.at[buf], rsem)

    def wait_rows(buf, src):
        pltpu.make_async_copy(
            src.at[cidx_v.at[buf]], rows_v.at[buf], rsem).wait()

    issue_idx(0, 0)
    wait_idx(0)
    issue_rows(0, x_sh)
    issue_idx(1, 1)

    def seg_body(i, cur_chunk):
        lo = ptr_s[i]
        hi = ptr_s[i + 1]
        c_lo = (lo - eb) // K
        c_hi = jnp.where(hi > lo, (hi - eb + (K - 1)) // K, c_lo)

        def chunk_body(c, carry):
            cur = carry[0]
            acc = list(carry[1:])
            par = c & 1
            nxt = 1 - par

            @pl.when(c != cur)
            def _():
                @pl.when(par == 0)
                def _():
                    wait_rows(par, x_sh)
                    wait_idx(nxt)
                    issue_rows(nxt, x_hbm)

                @pl.when(par == 1)
                def _():
                    wait_rows(par, x_hbm)
                    wait_idx(nxt)
                    issue_rows(nxt, x_sh)

                issue_idx(c + 2, par)

            cb = eb + c * K
            j0 = jnp.maximum(lo - cb, 0)
            j1 = jnp.minimum(hi - cb, K)

            def edge_body(j, a):
                return tuple(
                    jnp.maximum(a[v], rows_v[par, j, pl.ds(v * L, L)])
                    for v in range(NV))

            acc = lax.fori_loop(j0, j1, edge_body, tuple(acc))
            return (c,) + tuple(acc)

        init = (cur_chunk,) + (_NEG_INF,) * NV
        fin = lax.fori_loop(c_lo, c_hi, chunk_body, init)

        obuf = (i // OB) & 1
        orow = i % OB
        for v in range(NV):
            blk_v[obuf, orow, pl.ds(v * L, L)] = jnp.where(
                fin[1 + v] > _NEG_INF, fin[1 + v], _ZERO)

        @pl.when(orow == OB - 1)
        def _():
            blkid = i // OB

            @pl.when(blkid >= 2)
            def _():
                pltpu.make_async_copy(
                    blk_v.at[0], out_hbm.at[pl.ds(s0, OB)], osem).wait()

            pltpu.async_copy(
                blk_v.at[obuf], out_hbm.at[pl.ds(s0 + blkid * OB, OB)], osem)

        return fin[0]

    cur = lax.fori_loop(0, SPW, seg_body, jnp.int32(-1))

    pe = (cur + 1) & 1
    wait_rows(pe, x_sh)
    wait_idx(1 - pe)
    pltpu.make_async_copy(blk_v.at[0], out_hbm.at[pl.ds(s0, OB)], osem).wait()
    pltpu.make_async_copy(blk_v.at[0], out_hbm.at[pl.ds(s0, OB)], osem).wait()


@jax.jit
def _launch(x, ptr_pad, idx_pad):
    mesh = plsc.VectorSubcoreMesh(core_axis_name="c", subcore_axis_name="s")
    f = pl.kernel(
        _body,
        mesh=mesh,
        out_type=jax.ShapeDtypeStruct((NPAD, D), jnp.float32),
        scratch_types=[
            pltpu.VMEM((PTR_BUF,), jnp.int32),
            pltpu.VMEM((2, K), jnp.int32),
            pltpu.VMEM((2, K, D), jnp.float32),
            pltpu.VMEM((2, OB, D), jnp.float32),
            pltpu.SMEM((PTR_BUF,), jnp.int32),
            pltpu.VMEM_SHARED((NPAD, D), jnp.float32),
            pltpu.SemaphoreType.DMA,
            pltpu.SemaphoreType.DMA,
            pltpu.SemaphoreType.DMA,
        ],
    )
    return f(x, ptr_pad, idx_pad)


def kernel(input, indptr, indices):
    x_pad = jnp.concatenate(
        [input, jnp.zeros((NPAD - N, D), dtype=input.dtype)])
    ptr_pad = jnp.concatenate(
        [indptr.astype(jnp.int32),
         jnp.full((PTR_PAD - (N + 1),), E, dtype=jnp.int32)])
    idx_pad = jnp.concatenate(
        [indices.astype(jnp.int32),
         jnp.zeros((E_PAD - E,), dtype=jnp.int32)])
    out_pad = _launch(x_pad, ptr_pad, idx_pad)
    return out_pad[:N]


# speedup vs baseline: 1.1786x; 1.1786x over previous
import functools

import jax
import jax.numpy as jnp
from jax import lax
from jax.experimental import pallas as pl
from jax.experimental.pallas import tpu as pltpu
from jax.experimental.pallas import tpu_sc as plsc

N = 10000
E = 320000
D = 128
L = 16
NV = D // L
NW = 32
SPW = 320
NPAD = NW * SPW
PTR_BUF = SPW + 16
PTR_PAD = (NW - 1) * SPW + PTR_BUF
K = 160
KH = K // 2
E_PAD = E + 4 * K
OB = 8


def _body(x_hbm, ptr_hbm, idx_hbm, out_hbm, ptr_v, cidx_v, rows_v, blk_v, ptr_s,
          x_sh, rsem, isem, osem):
    _NEG_INF = jnp.full((L,), -jnp.inf, dtype=jnp.float32)
    _ZERO = jnp.zeros((L,), dtype=jnp.float32)
    cid = lax.axis_index("c")
    sid = lax.axis_index("s")
    wid = sid * 2 + cid
    s0 = wid * SPW

    pltpu.sync_copy(ptr_hbm.at[pl.ds(s0, PTR_BUF)], ptr_v)
    for b in range(PTR_BUF // L):
        vec = ptr_v[pl.ds(b * L, L)]
        for l in range(L):
            ptr_s[b * L + l] = vec[l]

    RPS = NPAD // 16
    pltpu.sync_copy(x_hbm.at[pl.ds(sid * RPS, RPS)],
                    x_sh.at[pl.ds(sid * RPS, RPS)])
    plsc.subcore_barrier()

    e0 = ptr_s[0]
    eb = (e0 // 16) * 16

    def issue_idx(c, buf):
        for h in range(2):
            pltpu.async_copy(
                idx_hbm.at[pl.ds(eb + c * K + h * KH, KH)],
                cidx_v.at[buf, h], isem)

    def wait_idx(buf):
        for h in range(2):
            pltpu.make_async_copy(
                idx_hbm.at[pl.ds(eb, KH)], cidx_v.at[buf, h], isem).wait()

    def issue_rows(buf):
        for h in range(2):
            pltpu.async_copy(
                x_sh.at[cidx_v.at[buf, h]],
                rows_v.at[buf, pl.ds(h * KH, KH)], rsem)

    def wait_rows(buf):
        for h in range(2):
            pltpu.make_async_copy(
                x_sh.at[cidx_v.at[buf, h]],
                rows_v.at[buf, pl.ds(h * KH, KH)], rsem).wait()

    issue_idx(0, 0)
    wait_idx(0)
    issue_rows(0)
    issue_idx(1, 1)

    def seg_body(i, cur_chunk):
        lo = ptr_s[i]
        hi = ptr_s[i + 1]
        c_lo = (lo - eb) // K
        c_hi = jnp.where(hi > lo, (hi - eb + (K - 1)) // K, c_lo)

        def chunk_body(c, carry):
            cur = carry[0]
            acc = list(carry[1:])
            par = c & 1
            nxt = 1 - par

            @pl.when(c != cur)
            def _():
                wait_rows(par)
                wait_idx(nxt)
                issue_rows(nxt)
                issue_idx(c + 2, par)

            cb = eb + c * K
            j0 = jnp.maximum(lo - cb, 0)
            j1 = jnp.minimum(hi - cb, K)

            def edge_body(j, a):
                return tuple(
                    jnp.maximum(a[v], rows_v[par, j, pl.ds(v * L, L)])
                    for v in range(NV))

            acc = lax.fori_loop(j0, j1, edge_body, tuple(acc))
            return (c,) + tuple(acc)

        init = (cur_chunk,) + (_NEG_INF,) * NV
        fin = lax.fori_loop(c_lo, c_hi, chunk_body, init)

        obuf = (i // OB) & 1
        orow = i % OB
        for v in range(NV):
            blk_v[obuf, orow, pl.ds(v * L, L)] = jnp.where(
                fin[1 + v] > _NEG_INF, fin[1 + v], _ZERO)

        @pl.when(orow == OB - 1)
        def _():
            blkid = i // OB

            @pl.when(blkid >= 2)
            def _():
                pltpu.make_async_copy(
                    blk_v.at[0], out_hbm.at[pl.ds(s0, OB)], osem).wait()

            pltpu.async_copy(
                blk_v.at[obuf], out_hbm.at[pl.ds(s0 + blkid * OB, OB)], osem)

        return fin[0]

    cur = lax.fori_loop(0, SPW, seg_body, jnp.int32(-1))

    pe = (cur + 1) & 1
    wait_rows(pe)
    wait_idx(1 - pe)
    pltpu.make_async_copy(blk_v.at[0], out_hbm.at[pl.ds(s0, OB)], osem).wait()
    pltpu.make_async_copy(blk_v.at[0], out_hbm.at[pl.ds(s0, OB)], osem).wait()


@jax.jit
def _launch(x, ptr_pad, idx_pad):
    mesh = plsc.VectorSubcoreMesh(core_axis_name="c", subcore_axis_name="s")
    f = pl.kernel(
        _body,
        mesh=mesh,
        out_type=jax.ShapeDtypeStruct((NPAD, D), jnp.float32),
        scratch_types=[
            pltpu.VMEM((PTR_BUF,), jnp.int32),
            pltpu.VMEM((2, 2, KH), jnp.int32),
            pltpu.VMEM((2, K, D), jnp.float32),
            pltpu.VMEM((2, OB, D), jnp.float32),
            pltpu.SMEM((PTR_BUF,), jnp.int32),
            pltpu.VMEM_SHARED((NPAD, D), jnp.float32),
            pltpu.SemaphoreType.DMA,
            pltpu.SemaphoreType.DMA,
            pltpu.SemaphoreType.DMA,
        ],
    )
    return f(x, ptr_pad, idx_pad)


def kernel(input, indptr, indices):
    x_pad = jnp.concatenate(
        [input, jnp.zeros((NPAD - N, D), dtype=input.dtype)])
    ptr_pad = jnp.concatenate(
        [indptr.astype(jnp.int32),
         jnp.full((PTR_PAD - (N + 1),), E, dtype=jnp.int32)])
    idx_pad = jnp.concatenate(
        [indices.astype(jnp.int32),
         jnp.zeros((E_PAD - E,), dtype=jnp.int32)])
    out_pad = _launch(x_pad, ptr_pad, idx_pad)
    return out_pad[:N]


# speedup vs baseline: 1.2060x; 1.0233x over previous
import functools

import jax
import jax.numpy as jnp
from jax import lax
from jax.experimental import pallas as pl
from jax.experimental.pallas import tpu as pltpu
from jax.experimental.pallas import tpu_sc as plsc

N = 10000
E = 320000
D = 128
L = 16
NV = D // L
NW = 32
SPW = 320
NPAD = NW * SPW
PTR_BUF = SPW + 16
PTR_PAD = (NW - 1) * SPW + PTR_BUF
K = 128
E_PAD = E + 4 * K
OB = 8


def _body(x_hbm, ptr_hbm, idx_hbm, out_hbm, ptr_v, cidx_v, rows_v, blk_v, ptr_s,
          x_sh, rsem, isem, osem):
    _NEG_INF = jnp.full((L,), -jnp.inf, dtype=jnp.float32)
    _ZERO = jnp.zeros((L,), dtype=jnp.float32)
    cid = lax.axis_index("c")
    sid = lax.axis_index("s")
    wid = sid * 2 + cid
    s0 = wid * SPW

    pltpu.sync_copy(ptr_hbm.at[pl.ds(s0, PTR_BUF)], ptr_v)
    for b in range(PTR_BUF // L):
        vec = ptr_v[pl.ds(b * L, L)]
        for l in range(L):
            ptr_s[b * L + l] = vec[l]

    RPS = NPAD // 16
    pltpu.sync_copy(x_hbm.at[pl.ds(sid * RPS, RPS)],
                    x_sh.at[pl.ds(sid * RPS, RPS)])
    plsc.subcore_barrier()

    e0 = ptr_s[0]
    eb = (e0 // 16) * 16

    def issue_idx(c, buf):
        pltpu.async_copy(
            idx_hbm.at[pl.ds(eb + c * K, K)], cidx_v.at[buf], isem)

    def wait_idx(buf):
        pltpu.make_async_copy(
            idx_hbm.at[pl.ds(eb, K)], cidx_v.at[buf], isem).wait()

    def issue_rows(buf):
        pltpu.async_copy(x_sh.at[cidx_v.at[buf]], rows_v.at[buf], rsem)

    def wait_rows(buf):
        pltpu.make_async_copy(
            x_sh.at[cidx_v.at[buf]], rows_v.at[buf], rsem).wait()

    issue_idx(0, 0)
    wait_idx(0)
    issue_rows(0)
    issue_idx(1, 1)

    def seg_body(i, cur_chunk):
        lo = ptr_s[i]
        hi = ptr_s[i + 1]
        c_lo = (lo - eb) // K
        c_hi = jnp.where(hi > lo, (hi - eb + (K - 1)) // K, c_lo)

        def chunk_body(c, carry):
            cur = carry[0]
            acc = list(carry[1:])
            par = c & 1
            nxt = 1 - par

            @pl.when(c != cur)
            def _():
                wait_rows(par)
                wait_idx(nxt)
                issue_rows(nxt)
                issue_idx(c + 2, par)

            cb = eb + c * K
            j0 = jnp.maximum(lo - cb, 0)
            j1 = jnp.minimum(hi - cb, K)

            def edge_body(j, a):
                return tuple(
                    jnp.maximum(a[v], rows_v[par, j, pl.ds(v * L, L)])
                    for v in range(NV))

            acc = lax.fori_loop(j0, j1, edge_body, tuple(acc))
            return (c,) + tuple(acc)

        init = (cur_chunk,) + (_NEG_INF,) * NV
        fin = lax.fori_loop(c_lo, c_hi, chunk_body, init)

        obuf = (i // OB) & 1
        orow = i % OB
        for v in range(NV):
            blk_v[obuf, orow, pl.ds(v * L, L)] = jnp.where(
                fin[1 + v] > _NEG_INF, fin[1 + v], _ZERO)

        @pl.when(orow == OB - 1)
        def _():
            blkid = i // OB

            @pl.when(blkid >= 2)
            def _():
                pltpu.make_async_copy(
                    blk_v.at[0], out_hbm.at[pl.ds(s0, OB)], osem).wait()

            pltpu.async_copy(
                blk_v.at[obuf], out_hbm.at[pl.ds(s0 + blkid * OB, OB)], osem)

        return fin[0]

    cur = lax.fori_loop(0, SPW, seg_body, jnp.int32(-1))

    pe = (cur + 1) & 1
    wait_rows(pe)
    wait_idx(1 - pe)
    pltpu.make_async_copy(blk_v.at[0], out_hbm.at[pl.ds(s0, OB)], osem).wait()
    pltpu.make_async_copy(blk_v.at[0], out_hbm.at[pl.ds(s0, OB)], osem).wait()


@jax.jit
def _launch(x, ptr_pad, idx_pad):
    mesh = plsc.VectorSubcoreMesh(core_axis_name="c", subcore_axis_name="s")
    f = pl.kernel(
        _body,
        mesh=mesh,
        out_type=jax.ShapeDtypeStruct((NPAD, D), jnp.float32),
        scratch_types=[
            pltpu.VMEM((PTR_BUF,), jnp.int32),
            pltpu.VMEM((2, K), jnp.int32),
            pltpu.VMEM((2, K, D), jnp.float32),
            pltpu.VMEM((2, OB, D), jnp.float32),
            pltpu.SMEM((PTR_BUF,), jnp.int32),
            pltpu.VMEM_SHARED((NPAD, D), jnp.float32),
            pltpu.SemaphoreType.DMA,
            pltpu.SemaphoreType.DMA,
            pltpu.SemaphoreType.DMA,
        ],
    )
    return f(x, ptr_pad, idx_pad)


def kernel(input, indptr, indices):
    x_pad = jnp.concatenate(
        [input, jnp.zeros((NPAD - N, D), dtype=input.dtype)])
    ptr_pad = jnp.concatenate(
        [indptr.astype(jnp.int32),
         jnp.full((PTR_PAD - (N + 1),), E, dtype=jnp.int32)])
    idx_pad = jnp.concatenate(
        [indices.astype(jnp.int32),
         jnp.zeros((E_PAD - E,), dtype=jnp.int32)])
    out_pad = _launch(x_pad, ptr_pad, idx_pad)
    return out_pad[:N]
